# baseline (speedup 1.0000x reference)
import functools

import jax
import jax.numpy as jnp
from jax.experimental import pallas as pl
from jax.experimental.pallas import tpu as pltpu


def _softplus(x):
    return jnp.maximum(x, 0.0) + jnp.log1p(jnp.exp(-jnp.abs(x)))


def _flow_body(fwd_ref, bwd_ref, w1_ref, b1_ref, w2_ref, b2_ref, out_ref, *, nact, d):
    w1 = w1_ref[...]
    b1 = b1_ref[...]
    w2 = w2_ref[...]
    b2 = b2_ref[...]

    xf = fwd_ref[...]
    hf = jnp.maximum(jnp.dot(xf, w1, preferred_element_type=jnp.float32) + b1, 0.0)
    zf = jnp.dot(hf, w2, preferred_element_type=jnp.float32) + b2
    fout = jnp.sum(_softplus(zf), axis=1, keepdims=True)

    fin = jnp.zeros_like(fout)
    for i in range(nact):
        xb = bwd_ref[:, (i + 1) * d : (i + 2) * d]
        hb = jnp.maximum(jnp.dot(xb, w1, preferred_element_type=jnp.float32) + b1, 0.0)
        zb = jnp.dot(hb, w2, preferred_element_type=jnp.float32) + b2
        fin = fin + _softplus(zb[:, i : i + 1])

    out_ref[:, 0:1] = fin
    out_ref[:, 1:2] = fout


@functools.partial(jax.jit, static_argnames=("interpret",))
def _flow_pallas(forward_edges, backward_edges, W1, b1, W2, b2, interpret=False):
    b, t, a1, d = forward_edges.shape
    nact = a1 - 1
    h = W1.shape[1]
    n = b * t
    blk = 512

    fwd = forward_edges.reshape(n, a1 * d)
    bwd = backward_edges.reshape(n, a1 * d)

    out = pl.pallas_call(
        functools.partial(_flow_body, nact=nact, d=d),
        grid=(n // blk,),
        in_specs=[
            pl.BlockSpec((blk, d), lambda i: (i, 0)),
            pl.BlockSpec((blk, a1 * d), lambda i: (i, 0)),
            pl.BlockSpec((d, h), lambda i: (0, 0)),
            pl.BlockSpec((1, h), lambda i: (0, 0)),
            pl.BlockSpec((h, nact), lambda i: (0, 0)),
            pl.BlockSpec((1, nact), lambda i: (0, 0)),
        ],
        out_specs=pl.BlockSpec((blk, 2), lambda i: (i, 0)),
        out_shape=jax.ShapeDtypeStruct((n, 2), jnp.float32),
        compiler_params=pltpu.CompilerParams(
            dimension_semantics=("parallel",),
        ),
        interpret=interpret,
    )(fwd, bwd, W1, b1.reshape(1, h), W2, b2.reshape(1, nact))
    return out


def kernel(forward_edges, backward_edges, paths_reward, W1, b1, W2, b2, initial_flow):
    b, t, a1, d = forward_edges.shape
    out = _flow_pallas(forward_edges, backward_edges, W1, b1, W2, b2)
    fin_fout = out.reshape(b, t, 2)
    r = paths_reward.reshape(b, t, 1)
    finit = jnp.broadcast_to(initial_flow.reshape(1, 1, 1), (b, t, 1)).astype(jnp.float32)
    return jnp.concatenate([fin_fout, r, finit], axis=-1)


# speedup vs baseline: 1.9501x; 1.9501x over previous
import functools

import jax
import jax.numpy as jnp
from jax.experimental import pallas as pl
from jax.experimental.pallas import tpu as pltpu


def _softplus(x):
    return jnp.maximum(x, 0.0) + jnp.log1p(jnp.exp(-jnp.abs(x)))


def _flow_body(fwd_ref, bwd_ref, w1_ref, b1_ref, w2_ref, b2_ref, out_ref, *, nact, d):
    w1 = w1_ref[...]
    b1 = b1_ref[...]
    w2 = w2_ref[...]
    b2 = b2_ref[...]

    xf = fwd_ref[...]
    hf = jnp.maximum(jnp.dot(xf, w1, preferred_element_type=jnp.float32) + b1, 0.0)
    zf = jnp.dot(hf, w2, preferred_element_type=jnp.float32) + b2
    fout = jnp.sum(_softplus(zf), axis=1, keepdims=True)

    fin = jnp.zeros_like(fout)
    for i in range(nact):
        xb = bwd_ref[:, i + 1, :]
        hb = jnp.maximum(jnp.dot(xb, w1, preferred_element_type=jnp.float32) + b1, 0.0)
        zb = jnp.dot(hb, w2, preferred_element_type=jnp.float32) + b2
        fin = fin + _softplus(zb[:, i : i + 1])

    out_ref[:, 0:1] = fin
    out_ref[:, 1:2] = fout


@functools.partial(jax.jit, static_argnames=("interpret",))
def _flow_pallas(forward_edges, backward_edges, W1, b1, W2, b2, interpret=False):
    b, t, a1, d = forward_edges.shape
    nact = a1 - 1
    h = W1.shape[1]
    n = b * t
    blk = 512

    fwd = forward_edges[:, :, 0, :].reshape(n, d)
    bwd = backward_edges.reshape(n, a1, d)

    out = pl.pallas_call(
        functools.partial(_flow_body, nact=nact, d=d),
        grid=(n // blk,),
        in_specs=[
            pl.BlockSpec((blk, d), lambda i: (i, 0)),
            pl.BlockSpec((blk, a1, d), lambda i: (i, 0, 0)),
            pl.BlockSpec((d, h), lambda i: (0, 0)),
            pl.BlockSpec((1, h), lambda i: (0, 0)),
            pl.BlockSpec((h, nact), lambda i: (0, 0)),
            pl.BlockSpec((1, nact), lambda i: (0, 0)),
        ],
        out_specs=pl.BlockSpec((blk, 2), lambda i: (i, 0)),
        out_shape=jax.ShapeDtypeStruct((n, 2), jnp.float32),
        compiler_params=pltpu.CompilerParams(
            dimension_semantics=("parallel",),
        ),
        interpret=interpret,
    )(fwd, bwd, W1, b1.reshape(1, h), W2, b2.reshape(1, nact))
    return out


def kernel(forward_edges, backward_edges, paths_reward, W1, b1, W2, b2, initial_flow):
    b, t, a1, d = forward_edges.shape
    out = _flow_pallas(forward_edges, backward_edges, W1, b1, W2, b2)
    fin_fout = out.reshape(b, t, 2)
    r = paths_reward.reshape(b, t, 1)
    finit = jnp.broadcast_to(initial_flow.reshape(1, 1, 1), (b, t, 1)).astype(jnp.float32)
    return jnp.concatenate([fin_fout, r, finit], axis=-1)
